# baseline (speedup 1.0000x reference)
import functools

import jax
import jax.numpy as jnp
from jax import lax
from jax.experimental import pallas as pl
from jax.experimental.pallas import tpu as pltpu
from jax.experimental.pallas import tpu_sc as plsc

S = 2048
D = 768
E = 64
I = 2048

TILE = 128
NT = 80
SPAD = NT * TILE

NC, NS = 2, 16
NW = NC * NS
RPW = S // NW


def _router_body(x_ref, gw_ref, pos_ref, w_ref, te_ref, na_ref):
    x = x_ref[...]
    gw = gw_ref[...]
    logits = lax.dot_general(x, gw, (((1,), (1,)), ((), ())),
                             preferred_element_type=jnp.float32)
    m = jnp.max(logits, axis=1, keepdims=True)
    w_ref[...] = 1.0 / jnp.sum(jnp.exp(logits - m), axis=1, keepdims=True)
    col = lax.broadcasted_iota(jnp.int32, (S, E), 1)
    e_idx = jnp.min(jnp.where(logits == m, col, E), axis=1, keepdims=True)
    ohf = (col == e_idx).astype(jnp.float32)
    G, R = 16, 128
    oh3 = ohf.reshape(G, R, E)
    lti = (lax.broadcasted_iota(jnp.int32, (R, R), 0)
           >= lax.broadcasted_iota(jnp.int32, (R, R), 1)).astype(jnp.float32)
    csum3 = lax.dot_general(jnp.broadcast_to(lti[None], (G, R, R)), oh3,
                            (((2,), (1,)), ((0,), (0,))),
                            preferred_element_type=jnp.float32)
    tot = jnp.sum(oh3, axis=1)
    ltg = (lax.broadcasted_iota(jnp.int32, (G, G), 0)
           > lax.broadcasted_iota(jnp.int32, (G, G), 1)).astype(jnp.float32)
    gp = lax.dot_general(ltg, tot, (((1,), (0,)), ((), ())),
                         preferred_element_type=jnp.float32)
    csum = (csum3 + gp[:, None, :]).reshape(S, E)
    rank = jnp.sum(ohf * csum, axis=1, keepdims=True) - 1.0
    counts = jnp.sum(ohf, axis=0, keepdims=True)
    tiles_e = jnp.floor((counts + (TILE - 1)) / TILE)
    uti = (lax.broadcasted_iota(jnp.int32, (E, E), 0)
           <= lax.broadcasted_iota(jnp.int32, (E, E), 1)).astype(jnp.float32)
    tinc = lax.dot_general(tiles_e, uti, (((1,), (0,)), ((), ())),
                           preferred_element_type=jnp.float32)
    poff = (tinc - tiles_e) * TILE
    pos_f = jnp.sum(ohf * poff, axis=1, keepdims=True) + rank
    pos_ref[...] = pos_f.astype(jnp.int32)
    na_ref[...] = tinc[:, E - 1:E].astype(jnp.int32)
    tj = lax.broadcasted_iota(jnp.int32, (NT, E), 0).astype(jnp.float32)
    owner = jnp.sum((jnp.broadcast_to(tinc, (NT, E)) <= tj).astype(jnp.float32),
                    axis=1, keepdims=True).astype(jnp.int32)
    te_ref[...] = jnp.minimum(owner, E - 1)


def _router(x2, gate_w):
    return pl.pallas_call(
        _router_body,
        out_shape=(
            jax.ShapeDtypeStruct((S, 1), jnp.int32),
            jax.ShapeDtypeStruct((S, 1), jnp.float32),
            jax.ShapeDtypeStruct((NT, 1), jnp.int32),
            jax.ShapeDtypeStruct((1, 1), jnp.int32),
        ),
    )(x2, gate_w)


@functools.lru_cache(maxsize=None)
def _sc_kernels():
    mesh = plsc.VectorSubcoreMesh(core_axis_name="c", subcore_axis_name="s")
    scratch = [
        pltpu.VMEM((RPW,), jnp.int32),
        pltpu.VMEM((RPW, D), jnp.float32),
        pltpu.SemaphoreType.DMA,
    ]

    @functools.partial(
        pl.kernel,
        mesh=mesh,
        out_type=jax.ShapeDtypeStruct((SPAD, D), jnp.float32),
        scratch_types=scratch,
    )
    def sc_scatter(x_hbm, pos_hbm, xs_hbm, idx_v, rows_v, sem):
        wid = lax.axis_index("s") * NC + lax.axis_index("c")
        base = wid * RPW
        pltpu.sync_copy(pos_hbm.at[pl.ds(base, RPW)], idx_v)
        pltpu.sync_copy(x_hbm.at[pl.ds(base, RPW)], rows_v)
        pltpu.async_copy(rows_v, xs_hbm.at[idx_v], sem).wait()

    @functools.partial(
        pl.kernel,
        mesh=mesh,
        out_type=jax.ShapeDtypeStruct((S, D), jnp.float32),
        scratch_types=scratch,
    )
    def sc_gather(ys_hbm, pos_hbm, yg_hbm, idx_v, rows_v, sem):
        wid = lax.axis_index("s") * NC + lax.axis_index("c")
        base = wid * RPW
        pltpu.sync_copy(pos_hbm.at[pl.ds(base, RPW)], idx_v)
        pltpu.async_copy(ys_hbm.at[idx_v], rows_v, sem).wait()
        pltpu.sync_copy(rows_v, yg_hbm.at[pl.ds(base, RPW)])

    return sc_scatter, sc_gather


NSHT = S // TILE


def _ffn(xt, g_ref, u_ref, d_ref):
    prec = lax.Precision.DEFAULT
    g = lax.dot_general(xt, g_ref[0], (((1,), (1,)), ((), ())),
                        preferred_element_type=jnp.float32,
                        precision=prec)
    u = lax.dot_general(xt, u_ref[0], (((1,), (1,)), ((), ())),
                        preferred_element_type=jnp.float32,
                        precision=prec)
    h = g * jax.nn.sigmoid(g) * u
    return lax.dot_general(h, d_ref[0], (((1,), (1,)), ((), ())),
                           preferred_element_type=jnp.float32,
                           precision=prec)


def _gemm_body(te_ref, na_ref, xs_ref, eg_ref, eu_ref, ed_ref,
               x_ref, sg_ref, su_ref, sd_ref, out_ref, sh_ref):
    j = pl.program_id(0)

    @pl.when(j < na_ref[0])
    def _():
        out_ref[...] = _ffn(xs_ref[...], eg_ref, eu_ref, ed_ref)

    @pl.when(j < NSHT)
    def _():
        sh_ref[...] = _ffn(x_ref[...], sg_ref, su_ref, sd_ref)


def _gemm(te, na, xs, eg, eu, ed, x2, sg, su, sd):
    sh_idx = lambda j, te, na: (jnp.minimum(j, NSHT - 1), 0)
    grid_spec = pltpu.PrefetchScalarGridSpec(
        num_scalar_prefetch=2,
        grid=(NT,),
        in_specs=[
            pl.BlockSpec((TILE, D), lambda j, te, na: (j, 0)),
            pl.BlockSpec((1, I, D), lambda j, te, na: (te[j], 0, 0)),
            pl.BlockSpec((1, I, D), lambda j, te, na: (te[j], 0, 0)),
            pl.BlockSpec((1, D, I), lambda j, te, na: (te[j], 0, 0)),
            pl.BlockSpec((TILE, D), sh_idx),
            pl.BlockSpec((1, I, D), lambda j, te, na: (0, 0, 0)),
            pl.BlockSpec((1, I, D), lambda j, te, na: (0, 0, 0)),
            pl.BlockSpec((1, D, I), lambda j, te, na: (0, 0, 0)),
        ],
        out_specs=(
            pl.BlockSpec((TILE, D), lambda j, te, na: (j, 0)),
            pl.BlockSpec((TILE, D), sh_idx),
        ),
    )
    return pl.pallas_call(
        _gemm_body,
        grid_spec=grid_spec,
        out_shape=(
            jax.ShapeDtypeStruct((SPAD, D), jnp.float32),
            jax.ShapeDtypeStruct((S, D), jnp.float32),
        ),
    )(te, na, xs, eg, eu, ed, x2, sg, su, sd)


T2 = 256


def _combine_body(sh_ref, yg_ref, w_ref, out_ref):
    out_ref[...] = sh_ref[...] + yg_ref[...] * w_ref[...]


def _combine(sh, yg, w2):
    return pl.pallas_call(
        _combine_body,
        grid=(S // T2,),
        in_specs=[
            pl.BlockSpec((T2, D), lambda j: (j, 0)),
            pl.BlockSpec((T2, D), lambda j: (j, 0)),
            pl.BlockSpec((T2, 1), lambda j: (j, 0)),
        ],
        out_specs=pl.BlockSpec((T2, D), lambda j: (j, 0)),
        out_shape=jax.ShapeDtypeStruct((S, D), jnp.float32),
    )(sh, yg, w2)


def kernel(x, gate_w, eg, eu, ed, sg, su, sd):
    B, seq, d = x.shape
    x2 = x.reshape(S, D)
    pos2, w2, te2, na2 = _router(x2, gate_w)
    pos = pos2.reshape(S)
    te = te2.reshape(NT)
    na = na2.reshape(1)
    sc_scatter, sc_gather = _sc_kernels()
    xs = sc_scatter(x2, pos)
    ys, sh = _gemm(te, na, xs, eg, eu, ed, x2, sg, su, sd)
    yg = sc_gather(ys, pos)
    y = _combine(sh, yg, w2)
    return y.reshape(B, seq, d)


# speedup vs baseline: 1.0323x; 1.0323x over previous
import functools

import jax
import jax.numpy as jnp
from jax import lax
from jax.experimental import pallas as pl
from jax.experimental.pallas import tpu as pltpu
from jax.experimental.pallas import tpu_sc as plsc

S = 2048
D = 768
E = 64
I = 2048

TILE = 128
NT = 80
SPAD = NT * TILE

NC, NS = 2, 16
NW = NC * NS
RPW = S // NW


def _router_body(x_ref, gw_ref, pos_ref, w_ref, te_ref, na_ref):
    x = x_ref[...]
    gw = gw_ref[...]
    logits = lax.dot_general(x, gw, (((1,), (1,)), ((), ())),
                             preferred_element_type=jnp.float32)
    m = jnp.max(logits, axis=1, keepdims=True)
    w_ref[...] = 1.0 / jnp.sum(jnp.exp(logits - m), axis=1, keepdims=True)
    col = lax.broadcasted_iota(jnp.int32, (S, E), 1)
    e_idx = jnp.min(jnp.where(logits == m, col, E), axis=1, keepdims=True)
    ohf = (col == e_idx).astype(jnp.float32)
    G, R = 16, 128
    oh3 = ohf.reshape(G, R, E)
    lti = (lax.broadcasted_iota(jnp.int32, (R, R), 0)
           >= lax.broadcasted_iota(jnp.int32, (R, R), 1)).astype(jnp.float32)
    csum3 = lax.dot_general(jnp.broadcast_to(lti[None], (G, R, R)), oh3,
                            (((2,), (1,)), ((0,), (0,))),
                            preferred_element_type=jnp.float32)
    tot = jnp.sum(oh3, axis=1)
    ltg = (lax.broadcasted_iota(jnp.int32, (G, G), 0)
           > lax.broadcasted_iota(jnp.int32, (G, G), 1)).astype(jnp.float32)
    gp = lax.dot_general(ltg, tot, (((1,), (0,)), ((), ())),
                         preferred_element_type=jnp.float32)
    csum = (csum3 + gp[:, None, :]).reshape(S, E)
    rank = jnp.sum(ohf * csum, axis=1, keepdims=True) - 1.0
    counts = jnp.sum(ohf, axis=0, keepdims=True)
    tiles_e = jnp.floor((counts + (TILE - 1)) / TILE)
    uti = (lax.broadcasted_iota(jnp.int32, (E, E), 0)
           <= lax.broadcasted_iota(jnp.int32, (E, E), 1)).astype(jnp.float32)
    tinc = lax.dot_general(tiles_e, uti, (((1,), (0,)), ((), ())),
                           preferred_element_type=jnp.float32)
    poff = (tinc - tiles_e) * TILE
    pos_f = jnp.sum(ohf * poff, axis=1, keepdims=True) + rank
    pos_ref[...] = pos_f.astype(jnp.int32)
    na_ref[...] = tinc[:, E - 1:E].astype(jnp.int32)
    tj = lax.broadcasted_iota(jnp.int32, (NT, E), 0).astype(jnp.float32)
    owner = jnp.sum((jnp.broadcast_to(tinc, (NT, E)) <= tj).astype(jnp.float32),
                    axis=1, keepdims=True).astype(jnp.int32)
    te_ref[...] = jnp.minimum(owner, E - 1)


def _router(x2, gate_w):
    return pl.pallas_call(
        _router_body,
        out_shape=(
            jax.ShapeDtypeStruct((S, 1), jnp.int32),
            jax.ShapeDtypeStruct((S, 1), jnp.float32),
            jax.ShapeDtypeStruct((NT, 1), jnp.int32),
            jax.ShapeDtypeStruct((1, 1), jnp.int32),
        ),
    )(x2, gate_w)


@functools.lru_cache(maxsize=None)
def _sc_kernels():
    mesh = plsc.VectorSubcoreMesh(core_axis_name="c", subcore_axis_name="s")
    scratch = [
        pltpu.VMEM((RPW,), jnp.int32),
        pltpu.VMEM((RPW, D), jnp.float32),
        pltpu.SemaphoreType.DMA,
    ]

    @functools.partial(
        pl.kernel,
        mesh=mesh,
        out_type=jax.ShapeDtypeStruct((SPAD, D), jnp.float32),
        scratch_types=scratch,
    )
    def sc_scatter(x_hbm, pos_hbm, xs_hbm, idx_v, rows_v, sem):
        wid = lax.axis_index("s") * NC + lax.axis_index("c")
        base = wid * RPW
        pltpu.sync_copy(pos_hbm.at[pl.ds(base, RPW)], idx_v)
        pltpu.sync_copy(x_hbm.at[pl.ds(base, RPW)], rows_v)
        pltpu.async_copy(rows_v, xs_hbm.at[idx_v], sem).wait()

    @functools.partial(
        pl.kernel,
        mesh=mesh,
        out_type=jax.ShapeDtypeStruct((S, D), jnp.float32),
        scratch_types=scratch,
    )
    def sc_gather(ys_hbm, pos_hbm, yg_hbm, idx_v, rows_v, sem):
        wid = lax.axis_index("s") * NC + lax.axis_index("c")
        base = wid * RPW
        pltpu.sync_copy(pos_hbm.at[pl.ds(base, RPW)], idx_v)
        pltpu.async_copy(ys_hbm.at[idx_v], rows_v, sem).wait()
        pltpu.sync_copy(rows_v, yg_hbm.at[pl.ds(base, RPW)])

    return sc_scatter, sc_gather


NSHT = S // TILE


def _ffn(xt, g_ref, u_ref, d_ref):
    g = lax.dot_general(xt, g_ref[0], (((1,), (1,)), ((), ())),
                        preferred_element_type=jnp.float32)
    u = lax.dot_general(xt, u_ref[0], (((1,), (1,)), ((), ())),
                        preferred_element_type=jnp.float32)
    h = g * jax.nn.sigmoid(g) * u
    return lax.dot_general(h, d_ref[0], (((1,), (1,)), ((), ())),
                           preferred_element_type=jnp.float32)


SCH = 4
CH = I // SCH


def _gemm_body(te_ref, na_ref, xs_ref, eg_ref, eu_ref, ed_ref,
               x_ref, sg_ref, su_ref, sd4_ref, out_ref, sh_ref):
    j = pl.program_id(0)

    @pl.when(j < na_ref[0])
    def _():
        out_ref[...] = _ffn(xs_ref[...], eg_ref, eu_ref, ed_ref)

    @pl.when(j < SCH * NSHT)
    def _():
        c = j % SCH
        xt = x_ref[...]
        sgc = sg_ref[0, pl.ds(c * CH, CH), :]
        suc = su_ref[0, pl.ds(c * CH, CH), :]
        g = lax.dot_general(xt, sgc, (((1,), (1,)), ((), ())),
                            preferred_element_type=jnp.float32)
        u = lax.dot_general(xt, suc, (((1,), (1,)), ((), ())),
                            preferred_element_type=jnp.float32)
        h = g * jax.nn.sigmoid(g) * u
        sdc = sd4_ref[0, c]
        part = lax.dot_general(h, sdc, (((1,), (1,)), ((), ())),
                               preferred_element_type=jnp.float32)

        @pl.when(c == 0)
        def _():
            sh_ref[...] = part

        @pl.when(c != 0)
        def _():
            sh_ref[...] += part


def _gemm(te, na, xs, eg, eu, ed, x2, sg, su, sd4):
    sh_idx = lambda j, te, na: (jnp.minimum(j // SCH, NSHT - 1), 0)
    grid_spec = pltpu.PrefetchScalarGridSpec(
        num_scalar_prefetch=2,
        grid=(NT,),
        in_specs=[
            pl.BlockSpec((TILE, D),
                         lambda j, te, na: (jnp.minimum(j, na[0] - 1), 0)),
            pl.BlockSpec((1, I, D), lambda j, te, na: (te[j], 0, 0)),
            pl.BlockSpec((1, I, D), lambda j, te, na: (te[j], 0, 0)),
            pl.BlockSpec((1, D, I), lambda j, te, na: (te[j], 0, 0)),
            pl.BlockSpec((TILE, D), sh_idx),
            pl.BlockSpec((1, I, D), lambda j, te, na: (0, 0, 0)),
            pl.BlockSpec((1, I, D), lambda j, te, na: (0, 0, 0)),
            pl.BlockSpec((1, SCH, D, CH), lambda j, te, na: (0, 0, 0, 0)),
        ],
        out_specs=(
            pl.BlockSpec((TILE, D), lambda j, te, na: (j, 0)),
            pl.BlockSpec((TILE, D), sh_idx),
        ),
    )
    return pl.pallas_call(
        _gemm_body,
        grid_spec=grid_spec,
        out_shape=(
            jax.ShapeDtypeStruct((SPAD, D), jnp.float32),
            jax.ShapeDtypeStruct((S, D), jnp.float32),
        ),
    )(te, na, xs, eg, eu, ed, x2, sg, su, sd4)


T2 = 256


def _combine_body(sh_ref, yg_ref, w_ref, out_ref):
    out_ref[...] = sh_ref[...] + yg_ref[...] * w_ref[...]


def _combine(sh, yg, w2):
    return pl.pallas_call(
        _combine_body,
        grid=(S // T2,),
        in_specs=[
            pl.BlockSpec((T2, D), lambda j: (j, 0)),
            pl.BlockSpec((T2, D), lambda j: (j, 0)),
            pl.BlockSpec((T2, 1), lambda j: (j, 0)),
        ],
        out_specs=pl.BlockSpec((T2, D), lambda j: (j, 0)),
        out_shape=jax.ShapeDtypeStruct((S, D), jnp.float32),
    )(sh, yg, w2)


def kernel(x, gate_w, eg, eu, ed, sg, su, sd):
    B, seq, d = x.shape
    x2 = x.reshape(S, D)
    pos2, w2, te2, na2 = _router(x2, gate_w)
    pos = pos2.reshape(S)
    te = te2.reshape(NT)
    na = na2.reshape(1)
    sc_scatter, sc_gather = _sc_kernels()
    xs = sc_scatter(x2, pos)
    sd4 = sd.reshape(1, D, SCH, CH).transpose(0, 2, 1, 3)
    ys, sh = _gemm(te, na, xs, eg, eu, ed, x2, sg, su, sd4)
    yg = sc_gather(ys, pos)
    y = _combine(sh, yg, w2)
    return y.reshape(B, seq, d)
